# baseline (speedup 1.0000x reference)
import functools

import jax
import jax.numpy as jnp
from jax import lax
from jax.experimental import pallas as pl
from jax.experimental.pallas import tpu as pltpu
from jax.experimental.pallas import tpu_sc as plsc


@functools.lru_cache(maxsize=None)
def _make_gather(V, D, B):
    info = plsc.get_sparse_core_info()
    nc, ns = info.num_cores, info.num_subcores
    nw = nc * ns
    n_chunks = B // 16
    n_coltiles = (V + 127) // 128
    nt_pad = 256
    assert (n_coltiles + nw - 1) // nw <= nt_pad
    cap = 2 * (B // nw)
    nband = D // 8
    mesh = plsc.VectorSubcoreMesh(core_axis_name="c", subcore_axis_name="s")

    @functools.partial(
        pl.kernel,
        mesh=mesh,
        out_type=jax.ShapeDtypeStruct((B * D,), jnp.float32),
        scratch_types=[
            pltpu.VMEM((B,), jnp.int32),
            pltpu.VMEM((nt_pad,), jnp.int32),
            pltpu.VMEM((nt_pad,), jnp.int32),
            pltpu.VMEM((nt_pad,), jnp.int32),
            pltpu.VMEM((cap,), jnp.int32),
            pltpu.VMEM((cap,), jnp.int32),
            pltpu.VMEM((nband, 8, 128), jnp.float32),
            pltpu.VMEM((16 * D,), jnp.float32),
            pltpu.SemaphoreType.DMA,
            pltpu.SemaphoreType.DMA,
        ],
        compiler_params=pltpu.CompilerParams(
            disable_bounds_checks=True, needs_layout_passes=False
        ),
    )
    def gather_kernel(
        idx_hbm, wt_hbm, out_hbm,
        idx_v, counts, offsets, arb, ord_val, ord_pos, stage_v, ring_v,
        sem, wsem,
    ):
        wid = lax.axis_index("s") * nc + lax.axis_index("c")
        lane = lax.iota(jnp.int32, 16)
        zeros16 = jnp.zeros((16,), jnp.int32)
        ones16 = jnp.full((16,), 1, jnp.int32)

        pltpu.sync_copy(idx_hbm, idx_v)
        for z in range(nt_pad // 16):
            counts[pl.ds(16 * z, 16)] = zeros16

        def count_body(g, cnt):
            v = idx_v[pl.ds(pl.multiple_of(g * 16, 16), 16)]
            c = v >> 7
            mine = (c % nw) == wid
            cl = c // nw
            plsc.addupdate_scatter(counts, [cl], ones16, mask=mine)
            npop = plsc.all_reduce_population_count(mine)
            return cnt + npop[0]

        cnt = lax.fori_loop(0, n_chunks, count_body, jnp.int32(0))

        def prefix_body(z, run):
            v = counts[pl.ds(pl.multiple_of(16 * z, 16), 16)]
            pc = plsc.cumsum(v)
            offsets[pl.ds(pl.multiple_of(16 * z, 16), 16)] = pc - v + run
            return run + pc[15]

        lax.fori_loop(0, nt_pad // 16, prefix_body, jnp.int32(0))

        def scatter_body(g, carry):
            v = idx_v[pl.ds(pl.multiple_of(g * 16, 16), 16)]
            kvec = jnp.full((16,), g * 16, jnp.int32) + lane
            c = v >> 7
            mine = (c % nw) == wid
            cl = c // nw

            def w_cond(m):
                return jnp.max(m.astype(jnp.int32)) > 0

            def w_body(m):
                plsc.store_scatter(arb, [cl], lane, mask=m)
                got = plsc.load_gather(arb, [cl])
                win = jnp.logical_and(m, got == lane)
                offs = plsc.load_gather(offsets, [cl])
                slot = jnp.minimum(offs, cap - 1)
                plsc.store_scatter(ord_val, [slot], v, mask=win)
                plsc.store_scatter(ord_pos, [slot], kvec, mask=win)
                plsc.store_scatter(offsets, [cl], offs + 1, mask=win)
                return jnp.logical_and(m, jnp.logical_not(win))

            lax.while_loop(w_cond, w_body, mine)
            return carry

        lax.fori_loop(0, n_chunks, scatter_body, jnp.int32(0))

        band_idx = [(16 * g2 + lane) >> 3 for g2 in range(D // 16)]
        sub_idx = [(16 * g2 + lane) & 7 for g2 in range(D // 16)]

        def fetch(c):
            c0 = pl.multiple_of(c * 128, 128)
            for r in range(nband):
                pltpu.async_copy(
                    wt_hbm.at[pl.ds(8 * r, 8), pl.ds(c0, 128)],
                    stage_v.at[r],
                    sem,
                )
            for r in range(nband):
                pltpu.make_async_copy(
                    wt_hbm.at[pl.ds(0, 8), pl.ds(0, 128)],
                    stage_v.at[r],
                    sem,
                ).wait()

        n_groups = (cnt + 15) >> 4

        def walk_body(g, c_last):
            gbase = pl.multiple_of(g * 16, 16)
            v16 = ord_val[pl.ds(gbase, 16)]
            p16 = ord_pos[pl.ds(gbase, 16)]
            cl_ = c_last
            for lj in range(16):
                j = g * 16 + lj
                val = v16[lj]
                c = val >> 7
                l = val & 127
                valid = j < cnt
                new_c = jnp.logical_and(valid, c != cl_)

                @pl.when(new_c)
                def _():
                    fetch(c)

                @pl.when(valid)
                def _():
                    @pl.when(j >= 16)
                    def _():
                        pltpu.make_async_copy(
                            out_hbm.at[pl.ds(0, D)],
                            ring_v.at[pl.ds(0, D)],
                            wsem,
                        ).wait()

                    lvec = jnp.full((16,), l, jnp.int32)
                    for g2 in range(D // 16):
                        vals = plsc.load_gather(
                            stage_v, [band_idx[g2], sub_idx[g2], lvec]
                        )
                        ring_v[pl.ds(lj * D + 16 * g2, 16)] = vals

                    k = p16[lj]
                    pltpu.async_copy(
                        ring_v.at[pl.ds(lj * D, D)],
                        out_hbm.at[pl.ds(pl.multiple_of(k * D, 8), D)],
                        wsem,
                    )

                cl_ = jnp.where(valid, c, cl_)
            return cl_

        lax.fori_loop(0, n_groups, walk_body, jnp.int32(-1))

        def drain_body(g, carry):
            pltpu.make_async_copy(
                out_hbm.at[pl.ds(0, D)], ring_v.at[pl.ds(0, D)], wsem
            ).wait()
            return carry

        lax.fori_loop(0, jnp.minimum(cnt, 16), drain_body, jnp.int32(0))

    return gather_kernel


def kernel(x, weight):
    (B,) = x.shape
    V, D = weight.shape
    fn = _make_gather(V, D, B)
    idx = x.astype(jnp.int32)
    flat = fn(idx, weight.T)
    return flat.reshape(B, D)


# speedup vs baseline: 1.0249x; 1.0249x over previous
import functools

import jax
import jax.numpy as jnp
from jax import lax
from jax.experimental import pallas as pl
from jax.experimental.pallas import tpu as pltpu
from jax.experimental.pallas import tpu_sc as plsc


@functools.lru_cache(maxsize=None)
def _make_gather(V, D, B):
    info = plsc.get_sparse_core_info()
    nc, ns = info.num_cores, info.num_subcores
    nw = nc * ns
    n_chunks = B // 16
    n_coltiles = (V + 127) // 128
    nt_pad = 256
    assert (n_coltiles + nw - 1) // nw <= nt_pad
    cap = 2 * (B // nw)
    nband = D // 8
    mesh = plsc.VectorSubcoreMesh(core_axis_name="c", subcore_axis_name="s")

    @functools.partial(
        pl.kernel,
        mesh=mesh,
        out_type=jax.ShapeDtypeStruct((B * D,), jnp.float32),
        scratch_types=[
            pltpu.VMEM((B,), jnp.int32),
            pltpu.VMEM((nt_pad,), jnp.int32),
            pltpu.VMEM((nt_pad,), jnp.int32),
            pltpu.VMEM((nt_pad,), jnp.int32),
            pltpu.VMEM((cap,), jnp.int32),
            pltpu.VMEM((cap,), jnp.int32),
            pltpu.VMEM((2, nband, 8, 128), jnp.float32),
            pltpu.VMEM((16 * D,), jnp.float32),
            pltpu.SemaphoreType.DMA,
            pltpu.SemaphoreType.DMA,
            pltpu.SemaphoreType.DMA,
        ],
        compiler_params=pltpu.CompilerParams(
            disable_bounds_checks=True, needs_layout_passes=False
        ),
    )
    def gather_kernel(
        idx_hbm, wt_hbm, out_hbm,
        idx_v, counts, offsets, arb, ord_val, ord_pos, stage_v, ring_v,
        sem0, sem1, wsem,
    ):
        wid = lax.axis_index("s") * nc + lax.axis_index("c")
        lane = lax.iota(jnp.int32, 16)
        zeros16 = jnp.zeros((16,), jnp.int32)
        ones16 = jnp.full((16,), 1, jnp.int32)

        pltpu.sync_copy(idx_hbm, idx_v)
        for z in range(nt_pad // 16):
            counts[pl.ds(16 * z, 16)] = zeros16

        def count_body(g, cnt):
            v = idx_v[pl.ds(pl.multiple_of(g * 16, 16), 16)]
            c = v >> 7
            mine = (c % nw) == wid
            cl = c // nw
            plsc.addupdate_scatter(counts, [cl], ones16, mask=mine)
            npop = plsc.all_reduce_population_count(mine)
            return cnt + npop[0]

        cnt = lax.fori_loop(0, n_chunks, count_body, jnp.int32(0))

        def prefix_body(z, run):
            v = counts[pl.ds(pl.multiple_of(16 * z, 16), 16)]
            pc = plsc.cumsum(v)
            offsets[pl.ds(pl.multiple_of(16 * z, 16), 16)] = pc - v + run
            return run + pc[15]

        lax.fori_loop(0, nt_pad // 16, prefix_body, jnp.int32(0))

        def scatter_body(g, carry):
            v = idx_v[pl.ds(pl.multiple_of(g * 16, 16), 16)]
            kvec = jnp.full((16,), g * 16, jnp.int32) + lane
            c = v >> 7
            mine = (c % nw) == wid
            cl = c // nw

            def w_cond(m):
                return jnp.max(m.astype(jnp.int32)) > 0

            def w_body(m):
                plsc.store_scatter(arb, [cl], lane, mask=m)
                got = plsc.load_gather(arb, [cl])
                win = jnp.logical_and(m, got == lane)
                offs = plsc.load_gather(offsets, [cl])
                slot = jnp.minimum(offs, cap - 1)
                plsc.store_scatter(ord_val, [slot], v, mask=win)
                plsc.store_scatter(ord_pos, [slot], kvec, mask=win)
                plsc.store_scatter(offsets, [cl], offs + 1, mask=win)
                return jnp.logical_and(m, jnp.logical_not(win))

            lax.while_loop(w_cond, w_body, mine)
            return carry

        lax.fori_loop(0, n_chunks, scatter_body, jnp.int32(0))

        band_idx = [(16 * g2 + lane) >> 3 for g2 in range(D // 16)]
        sub_idx = [(16 * g2 + lane) & 7 for g2 in range(D // 16)]

        def fire(c, b, s):
            c0 = pl.multiple_of(c * 128, 128)
            for r in range(nband):
                pltpu.async_copy(
                    wt_hbm.at[pl.ds(8 * r, 8), pl.ds(c0, 128)],
                    stage_v.at[b, r],
                    s,
                )

        def drain_tile(s):
            for r in range(nband):
                pltpu.make_async_copy(
                    wt_hbm.at[pl.ds(0, 8), pl.ds(0, 128)],
                    stage_v.at[0, r],
                    s,
                ).wait()

        n_groups = (cnt + 16) >> 4

        def walk_body(g, carry):
            c_f, par, owed0, owed1, l_prev, k_prev = carry
            gbase = pl.multiple_of(jnp.minimum(g, (cap // 16) - 1) * 16, 16)
            v16 = ord_val[pl.ds(gbase, 16)]
            p16 = ord_pos[pl.ds(gbase, 16)]
            for lj in range(16):
                j = g * 16 + lj
                val = v16[lj]
                c = val >> 7
                l = val & 127
                k = p16[lj]
                fetch_valid = j < cnt
                new_c = jnp.logical_and(fetch_valid, c != c_f)
                p_prev = par

                nb = 1 - par

                @pl.when(jnp.logical_and(new_c, nb == 0))
                def _():
                    fire(c, 0, sem0)

                @pl.when(jnp.logical_and(new_c, nb == 1))
                def _():
                    fire(c, 1, sem1)

                par = jnp.where(new_c, nb, par)
                owed0 = jnp.where(jnp.logical_and(new_c, nb == 0), 1, owed0)
                owed1 = jnp.where(jnp.logical_and(new_c, nb == 1), 1, owed1)
                c_f = jnp.where(new_c, c, c_f)

                ex_valid = jnp.logical_and(j >= 1, j <= cnt)
                owed_prev = jnp.where(p_prev == 0, owed0, owed1)

                @pl.when(
                    jnp.logical_and(
                        ex_valid,
                        jnp.logical_and(owed_prev == 1, p_prev == 0),
                    )
                )
                def _():
                    drain_tile(sem0)

                @pl.when(
                    jnp.logical_and(
                        ex_valid,
                        jnp.logical_and(owed_prev == 1, p_prev == 1),
                    )
                )
                def _():
                    drain_tile(sem1)

                owed0 = jnp.where(
                    jnp.logical_and(ex_valid, p_prev == 0), 0, owed0
                )
                owed1 = jnp.where(
                    jnp.logical_and(ex_valid, p_prev == 1), 0, owed1
                )

                @pl.when(ex_valid)
                def _():
                    @pl.when(j >= 17)
                    def _():
                        pltpu.make_async_copy(
                            out_hbm.at[pl.ds(0, D)],
                            ring_v.at[pl.ds(0, D)],
                            wsem,
                        ).wait()

                    lvec = jnp.full((16,), l_prev, jnp.int32)
                    pvec = jnp.full((16,), p_prev, jnp.int32)
                    for g2 in range(D // 16):
                        vals = plsc.load_gather(
                            stage_v,
                            [pvec, band_idx[g2], sub_idx[g2], lvec],
                        )
                        ring_v[pl.ds(lj * D + 16 * g2, 16)] = vals

                    pltpu.async_copy(
                        ring_v.at[pl.ds(lj * D, D)],
                        out_hbm.at[pl.ds(pl.multiple_of(k_prev * D, 8), D)],
                        wsem,
                    )

                l_prev = jnp.where(fetch_valid, l, l_prev)
                k_prev = jnp.where(fetch_valid, k, k_prev)
            return (c_f, par, owed0, owed1, l_prev, k_prev)

        lax.fori_loop(
            0,
            n_groups,
            walk_body,
            (
                jnp.int32(-1),
                jnp.int32(0),
                jnp.int32(0),
                jnp.int32(0),
                jnp.int32(0),
                jnp.int32(0),
            ),
        )

        def drain_body(g, carry):
            pltpu.make_async_copy(
                out_hbm.at[pl.ds(0, D)], ring_v.at[pl.ds(0, D)], wsem
            ).wait()
            return carry

        lax.fori_loop(0, jnp.minimum(cnt, 16), drain_body, jnp.int32(0))

    return gather_kernel


def kernel(x, weight):
    (B,) = x.shape
    V, D = weight.shape
    fn = _make_gather(V, D, B)
    idx = x.astype(jnp.int32)
    flat = fn(idx, weight.T)
    return flat.reshape(B, D)


# speedup vs baseline: 1.4331x; 1.3983x over previous
import functools

import jax
import jax.numpy as jnp
from jax import lax
from jax.experimental import pallas as pl
from jax.experimental.pallas import tpu as pltpu
from jax.experimental.pallas import tpu_sc as plsc


@functools.lru_cache(maxsize=None)
def _make_gather(V, D, B):
    info = plsc.get_sparse_core_info()
    nc, ns = info.num_cores, info.num_subcores
    nw = nc * ns
    n_chunks = B // 16
    n_coltiles = (V + 127) // 128
    nt_pad = 256
    assert (n_coltiles + nw - 1) // nw <= nt_pad
    cap = 2 * (B // nw)
    nband = D // 8
    mesh = plsc.VectorSubcoreMesh(core_axis_name="c", subcore_axis_name="s")

    @functools.partial(
        pl.kernel,
        mesh=mesh,
        out_type=jax.ShapeDtypeStruct((B * D,), jnp.float32),
        scratch_types=[
            pltpu.VMEM((B,), jnp.int32),
            pltpu.VMEM((nt_pad,), jnp.int32),
            pltpu.VMEM((nt_pad,), jnp.int32),
            pltpu.VMEM((nt_pad,), jnp.int32),
            pltpu.VMEM((cap + 16,), jnp.int32),
            pltpu.VMEM((cap + 16,), jnp.int32),
            pltpu.VMEM((8, nband, 8, 128), jnp.float32),
            pltpu.VMEM((16 * D,), jnp.float32),
            pltpu.SemaphoreType.DMA,
            pltpu.SemaphoreType.DMA,
            pltpu.SemaphoreType.DMA,
        ],
        compiler_params=pltpu.CompilerParams(
            disable_bounds_checks=True, needs_layout_passes=False
        ),
    )
    def gather_kernel(
        idx_hbm, wt_hbm, out_hbm,
        idx_v, counts, offsets, arb, ord_val, ord_pos, stage_v, ring_v,
        sem0, sem1, wsem,
    ):
        wid = lax.axis_index("s") * nc + lax.axis_index("c")
        lane = lax.iota(jnp.int32, 16)
        zeros16 = jnp.zeros((16,), jnp.int32)
        ones16 = jnp.full((16,), 1, jnp.int32)

        pltpu.sync_copy(idx_hbm, idx_v)
        for z in range(nt_pad // 16):
            counts[pl.ds(16 * z, 16)] = zeros16

        def count_body(g, cnt):
            v = idx_v[pl.ds(pl.multiple_of(g * 16, 16), 16)]
            c = v >> 7
            mine = (c % nw) == wid
            cl = c // nw
            plsc.addupdate_scatter(counts, [cl], ones16, mask=mine)
            npop = plsc.all_reduce_population_count(mine)
            return cnt + npop[0]

        cnt = lax.fori_loop(0, n_chunks, count_body, jnp.int32(0))

        def prefix_body(z, run):
            v = counts[pl.ds(pl.multiple_of(16 * z, 16), 16)]
            pc = plsc.cumsum(v)
            offsets[pl.ds(pl.multiple_of(16 * z, 16), 16)] = pc - v + run
            return run + pc[15]

        lax.fori_loop(0, nt_pad // 16, prefix_body, jnp.int32(0))

        def scatter_body(g, carry):
            v = idx_v[pl.ds(pl.multiple_of(g * 16, 16), 16)]
            kvec = jnp.full((16,), g * 16, jnp.int32) + lane
            c = v >> 7
            mine = (c % nw) == wid
            cl = c // nw

            def w_cond(m):
                return jnp.max(m.astype(jnp.int32)) > 0

            def w_body(m):
                plsc.store_scatter(arb, [cl], lane, mask=m)
                got = plsc.load_gather(arb, [cl])
                win = jnp.logical_and(m, got == lane)
                offs = plsc.load_gather(offsets, [cl])
                slot = jnp.minimum(offs, cap - 1)
                plsc.store_scatter(ord_val, [slot], v, mask=win)
                plsc.store_scatter(ord_pos, [slot], kvec, mask=win)
                plsc.store_scatter(offsets, [cl], offs + 1, mask=win)
                return jnp.logical_and(m, jnp.logical_not(win))

            lax.while_loop(w_cond, w_body, mine)
            return carry

        lax.fori_loop(0, n_chunks, scatter_body, jnp.int32(0))

        lane_h = lane >> 1
        prev_i = jnp.maximum(lane - 1, 0)
        lam_band = []
        lam_sub = []
        lam_off = []
        for a in range(D // 2):
            lam = 2 * a + (lane & 1)
            lam_band.append(lam >> 3)
            lam_sub.append(lam & 7)
            lam_off.append(lane_h * D + lam)

        def fire(c, b):
            c0 = pl.multiple_of(c * 128, 128)
            for r in range(nband):
                pltpu.async_copy(
                    wt_hbm.at[pl.ds(8 * r, 8), pl.ds(c0, 128)],
                    stage_v.at[b, r],
                    sem0,
                )

        n_groups = (cnt + 7) >> 3

        def wdrain(t, cc):
            pltpu.make_async_copy(
                out_hbm.at[pl.ds(0, D)], ring_v.at[pl.ds(0, D)], wsem
            ).wait()
            return cc

        def walk_body(g, carry):
            d2, d1 = carry
            gbase = pl.multiple_of(g * 8, 8)
            v16 = ord_val[pl.ds(gbase, 16)]
            p16 = ord_pos[pl.ds(gbase, 16)]
            c16 = v16 >> 7
            l16 = v16 & 127
            valid = (jnp.full((16,), g * 8, jnp.int32) + lane) < cnt
            in_grp = jnp.logical_and(valid, lane < 8)
            prevc = c16.at[prev_i].get(mode="promise_in_bounds")
            first = jnp.logical_and(
                in_grp, jnp.logical_or(c16 != prevc, lane == 0)
            )
            slot16 = plsc.cummax(jnp.where(first, lane, 0))
            n_fetch = plsc.all_reduce_population_count(first)[0]

            first_i = first.astype(jnp.int32)
            for i in range(8):

                @pl.when(first_i[i] == 1)
                def _():
                    fire(c16[i], i)

            lax.fori_loop(0, d2, wdrain, jnp.int32(0))

            def fdrain(t, cc):
                for r in range(nband):
                    pltpu.make_async_copy(
                        wt_hbm.at[pl.ds(0, 8), pl.ds(0, 128)],
                        stage_v.at[0, r],
                        sem0,
                    ).wait()
                return cc

            lax.fori_loop(0, n_fetch, fdrain, jnp.int32(0))

            slotv = slot16.at[lane_h].get(mode="promise_in_bounds")
            lv = l16.at[lane_h].get(mode="promise_in_bounds")
            vmask = (
                in_grp.astype(jnp.int32).at[lane_h].get(mode="promise_in_bounds")
                == 1
            )
            rbase = (g & 1) * (8 * D)
            for a in range(D // 2):
                vals = plsc.load_gather(
                    stage_v, [slotv, lam_band[a], lam_sub[a], lv]
                )
                ridx = jnp.full((16,), rbase, jnp.int32) + lam_off[a]
                plsc.store_scatter(ring_v, [ridx], vals, mask=vmask)

            in_grp_i = in_grp.astype(jnp.int32)
            for i in range(8):

                @pl.when(in_grp_i[i] == 1)
                def _():
                    pltpu.async_copy(
                        ring_v.at[pl.ds(pl.multiple_of(rbase + i * D, 8), D)],
                        out_hbm.at[pl.ds(pl.multiple_of(p16[i] * D, 8), D)],
                        wsem,
                    )

            dw = plsc.all_reduce_population_count(in_grp)[0]
            return (d1, dw)

        d2f, d1f = lax.fori_loop(
            0, n_groups, walk_body, (jnp.int32(0), jnp.int32(0))
        )

        lax.fori_loop(0, d2f + d1f, wdrain, jnp.int32(0))

    return gather_kernel


def kernel(x, weight):
    (B,) = x.shape
    V, D = weight.shape
    fn = _make_gather(V, D, B)
    idx = x.astype(jnp.int32)
    flat = fn(idx, weight.T)
    return flat.reshape(B, D)
